# baseline (speedup 1.0000x reference)
import functools

import jax
import jax.numpy as jnp
from jax import lax
from jax.experimental import pallas as pl
from jax.experimental.pallas import tpu as pltpu
from jax.experimental.pallas import tpu_sc as plsc

B = 16384
D = 32
L = 16
NC, NS = 2, 16
NW = NC * NS
BPW = B // NW
CH = 128
NCH = BPW // CH


def _body(o_hbm, a_hbm, s_hbm, b_hbm, res_hbm, idx_v, s_v, b_v, o_v, sem):
    wid = lax.axis_index("s") * NC + lax.axis_index("c")
    base = wid * BPW
    for j in range(NCH):
        pltpu.sync_copy(a_hbm.at[pl.ds(base + j * CH, CH)], idx_v.at[j])
    copies = []
    for j in range(NCH):
        copies.append(
            pltpu.async_copy(s_hbm.at[idx_v.at[j]], s_v.at[pl.ds(j * CH, CH)], sem))
        copies.append(
            pltpu.async_copy(b_hbm.at[idx_v.at[j]], b_v.at[pl.ds(j * CH, CH)], sem))
    pltpu.sync_copy(o_hbm.at[pl.ds(base, BPW)], o_v)
    for c in copies:
        c.wait()

    def fma_row(i, carry):
        for h in range(D // L):
            sl = pl.ds(h * L, L)
            o_v[i, sl] = s_v[i, sl] * o_v[i, sl] + b_v[i, sl]
        return carry

    lax.fori_loop(0, BPW, fma_row, 0)
    pltpu.sync_copy(o_v, res_hbm.at[pl.ds(base, BPW)])


def kernel(outputs, annotators, scale, bias):
    ann = annotators.astype(jnp.int32)
    mesh = plsc.VectorSubcoreMesh(core_axis_name="c", subcore_axis_name="s")
    k = functools.partial(
        pl.kernel,
        mesh=mesh,
        out_type=jax.ShapeDtypeStruct((B, D), jnp.float32),
        scratch_types=[
            pltpu.VMEM((NCH, CH), jnp.int32),
            pltpu.VMEM((BPW, D), jnp.float32),
            pltpu.VMEM((BPW, D), jnp.float32),
            pltpu.VMEM((BPW, D), jnp.float32),
            pltpu.SemaphoreType.DMA,
        ],
        compiler_params=pltpu.CompilerParams(use_tc_tiling_on_sc=False),
    )(_body)
    return k(outputs, ann, scale, bias)
